# baseline (speedup 1.0000x reference)
import functools

import numpy as np

import jax
import jax.numpy as jnp
from jax import lax
from jax.experimental import pallas as pl
from jax.experimental.pallas import tpu as pltpu
from jax.experimental.pallas import tpu_sc as plsc

B = 4
L = 4096
D = 1024
SCALE = float(D) ** 0.5

NC = 2
NS = 16
LANES = 16
NW = NC * NS
PPW = L // NW
C = 16
NPC = PPW // C
NSTEP = NPC * B
IPAD = PPW + LANES


def _sc_embed(table, xfp, pe):
    mesh = plsc.VectorSubcoreMesh(
        core_axis_name="c", subcore_axis_name="s", num_cores=NC, num_subcores=NS
    )

    @functools.partial(
        pl.kernel,
        out_type=jax.ShapeDtypeStruct((B * L, D), jnp.float32),
        mesh=mesh,
        scratch_types=(
            [pltpu.VMEM((IPAD,), jnp.int32) for _ in range(B)]
            + [pltpu.VMEM((C, D), jnp.float32) for _ in range(3)]
            + [pltpu.VMEM((C, D), jnp.float32) for _ in range(3)]
            + [pltpu.SemaphoreType.DMA for _ in range(9)]
        ),
    )
    def body(table_hbm, xfp_hbm, pe_hbm, out_hbm,
             i0, i1, i2, i3, r0, r1, r2, a0, a1, a2,
             g0, g1, g2, q0, q1, q2, w0, w1, w2):
        idx = [i0, i1, i2, i3]
        rows = [r0, r1, r2]
        acc = [a0, a1, a2]
        gsem = [g0, g1, g2]
        psem = [q0, q1, q2]
        wsem = [w0, w1, w2]

        wid = lax.axis_index("s") * NC + lax.axis_index("c")
        pbase = wid * PPW

        for b in range(B):
            pltpu.sync_copy(xfp_hbm.at[pl.ds(b * L + pbase, IPAD)], idx[b])

        def tok(s):
            return (s % B) * L + pbase + (s // B) * C

        def gather(s):
            pc, b = s // B, s % B
            return pltpu.async_copy(
                table_hbm.at[idx[b].at[pl.ds(pc * C, C)]], rows[s % 3], gsem[s % 3]
            )

        def pe_load(s):
            pc = s // B
            return pltpu.async_copy(
                pe_hbm.at[pl.ds(pbase + pc * C, C)], acc[s % 3], psem[s % 3]
            )

        gdesc = {0: gather(0), 1: gather(1)}
        pdesc = {0: pe_load(0), 1: pe_load(1)}
        wdesc = {}

        for s in range(NSTEP):
            pc, b = s // B, s % B
            if s + 2 < NSTEP:
                if s - 1 >= 0:
                    wdesc[(s + 2) % 3].wait()
                pdesc[(s + 2) % 3] = pe_load(s + 2)
                gdesc[(s + 2) % 3] = gather(s + 2)
            gdesc[s % 3].wait()
            pdesc[s % 3].wait()

            rv = rows[s % 3]
            av = acc[s % 3]
            ib = idx[b]

            def quarter_row(i, carry):
                t = i >> 2
                h = i & 3
                iv = ib[pl.ds(pc * C + t, LANES)][0]
                sv = jnp.where(iv != 0, jnp.float32(SCALE), jnp.float32(0.0))
                svv = jnp.full((LANES,), sv, jnp.float32)
                for j in range(D // (4 * LANES)):
                    dsl = pl.ds(h * (D // 4) + j * LANES, LANES)
                    plsc.addupdate(av.at[t, dsl], rv[t, dsl] * svv)
                return carry

            lax.fori_loop(0, 4 * C, quarter_row, 0)
            wdesc[s % 3] = pltpu.async_copy(av, out_hbm.at[pl.ds(tok(s), C)], wsem[s % 3])

        for s in range(NSTEP - 3, NSTEP):
            wdesc[s % 3].wait()

    return body(table, xfp, pe)


def _make_pe_rows():
    pos = np.arange(L, dtype=np.float32)[:, None].astype(np.float64)
    i = np.arange(D // 2, dtype=np.float32)[None, :].astype(np.float64)
    angle = (pos / np.power(10000.0, 2.0 * i / D)).astype(np.float32)
    pe = np.zeros((L, D), dtype=np.float32)
    pe[:, 0::2] = np.sin(angle, dtype=np.float32)
    pe[:, 1::2] = np.cos(angle, dtype=np.float32)
    return pe


_PE_ROWS = _make_pe_rows()


def kernel(x, embed_table):
    xf = x.reshape(B * L).astype(jnp.int32)
    xfp = jnp.concatenate([xf, jnp.zeros((LANES,), jnp.int32)])
    out = _sc_embed(embed_table, xfp, _PE_ROWS)
    return out.reshape(B, L, D)


# speedup vs baseline: 1.9863x; 1.9863x over previous
import functools

import numpy as np

import jax
import jax.numpy as jnp
from jax import lax
from jax.experimental import pallas as pl
from jax.experimental.pallas import tpu as pltpu
from jax.experimental.pallas import tpu_sc as plsc

B = 4
L = 4096
D = 1024
SCALE = float(D) ** 0.5

NC = 2
NS = 16
LANES = 16
NW = NC * NS
PPW = L // NW
C = 8
NSTEP = PPW // C
NR = 3
IPAD = PPW + LANES


def _sc_embed(table, xfp, pe):
    mesh = plsc.VectorSubcoreMesh(
        core_axis_name="c", subcore_axis_name="s", num_cores=NC, num_subcores=NS
    )

    @functools.partial(
        pl.kernel,
        out_type=jax.ShapeDtypeStruct((B * L, D), jnp.float32),
        mesh=mesh,
        scratch_types=(
            [pltpu.VMEM((IPAD,), jnp.int32) for _ in range(B)]
            + [pltpu.VMEM((C, D), jnp.float32) for _ in range(B * NR)]
            + [pltpu.VMEM((C, D), jnp.float32) for _ in range(2)]
            + [pltpu.SemaphoreType.DMA for _ in range(B * NR * 2 + 2)]
        ),
    )
    def body(table_hbm, xfp_hbm, pe_hbm, out_hbm, *scr):
        idx = list(scr[0:B])
        rows = [[scr[B + b * NR + k] for k in range(NR)] for b in range(B)]
        pev = list(scr[B + B * NR:B + B * NR + 2])
        ns = B + B * NR + 2
        gsem = [[scr[ns + b * NR + k] for k in range(NR)] for b in range(B)]
        wsem = [[scr[ns + B * NR + b * NR + k] for k in range(NR)] for b in range(B)]
        psem = list(scr[ns + 2 * B * NR:ns + 2 * B * NR + 2])

        wid = lax.axis_index("s") * NC + lax.axis_index("c")
        pbase = wid * PPW

        for b in range(B):
            pltpu.sync_copy(xfp_hbm.at[pl.ds(b * L + pbase, IPAD)], idx[b])

        def gather(pc, b):
            return pltpu.async_copy(
                table_hbm.at[idx[b].at[pl.ds(pc * C, C)]],
                rows[b][pc % NR], gsem[b][pc % NR]
            )

        def pe_load(pc):
            return pltpu.async_copy(
                pe_hbm.at[pl.ds(pbase + pc * C, C)], pev[pc % 2], psem[pc % 2]
            )

        pdesc = {0: pe_load(0), 1: pe_load(1)}
        gdesc = {}
        wdesc = {}
        for pc in range(2):
            for b in range(B):
                gdesc[(pc % NR, b)] = gather(pc, b)

        for pc in range(NSTEP):
            k = pc % NR
            if pc + 2 < NSTEP:
                k2 = (pc + 2) % NR
                if pc - 1 >= 0:
                    for b in range(B):
                        wdesc[(k2, b)].wait()
                for b in range(B):
                    gdesc[(k2, b)] = gather(pc + 2, b)
            if 2 <= pc + 1 < NSTEP:
                pdesc[(pc + 1) % 2] = pe_load(pc + 1)
            for b in range(B):
                gdesc[(k, b)].wait()
            pdesc[pc % 2].wait()

            rv = [rows[b][k] for b in range(B)]
            pv = pev[pc % 2]

            def quarter_row(i, carry):
                t = i >> 2
                h = i & 3
                svv = []
                for b in range(B):
                    iv = idx[b][pl.ds(pc * C + t, LANES)][0]
                    sv = jnp.where(iv != 0, jnp.float32(SCALE), jnp.float32(0.0))
                    svv.append(jnp.full((LANES,), sv, jnp.float32))
                for j in range(D // (4 * LANES)):
                    dsl = pl.ds(h * (D // 4) + j * LANES, LANES)
                    pvj = pv[t, dsl]
                    for b in range(B):
                        rv[b][t, dsl] = rv[b][t, dsl] * svv[b] + pvj
                return carry

            lax.fori_loop(0, 4 * C, quarter_row, 0)

            for b in range(B):
                wdesc[(k, b)] = pltpu.async_copy(
                    rv[b], out_hbm.at[pl.ds(b * L + pbase + pc * C, C)], wsem[b][k]
                )

        for pc in range(NSTEP - NR, NSTEP):
            for b in range(B):
                wdesc[(pc % NR, b)].wait()

    return body(table, xfp, pe)


def _make_pe_rows():
    pos = np.arange(L, dtype=np.float32)[:, None].astype(np.float64)
    i = np.arange(D // 2, dtype=np.float32)[None, :].astype(np.float64)
    angle = (pos / np.power(10000.0, 2.0 * i / D)).astype(np.float32)
    pe = np.zeros((L, D), dtype=np.float32)
    pe[:, 0::2] = np.sin(angle, dtype=np.float32)
    pe[:, 1::2] = np.cos(angle, dtype=np.float32)
    return pe


_PE_ROWS = _make_pe_rows()


def kernel(x, embed_table):
    xf = x.reshape(B * L).astype(jnp.int32)
    xfp = jnp.concatenate([xf, jnp.zeros((LANES,), jnp.int32)])
    out = _sc_embed(embed_table, xfp, _PE_ROWS)
    return out.reshape(B, L, D)
